# Initial kernel scaffold; baseline (speedup 1.0000x reference)
import jax
import jax.numpy as jnp
from jax.experimental import pallas as pl


def kernel(x, weight):
    raise NotImplementedError("write your pallas kernel here")


# speedup vs baseline: 1.7976x; 1.7976x over previous
import functools

import jax
import jax.numpy as jnp
from jax import lax
from jax.experimental import pallas as pl
from jax.experimental.pallas import tpu as pltpu
from jax.experimental.pallas import tpu_sc as plsc

_D = 64
_NC = 2
_NS = 16
_NW = _NC * _NS
_CHUNK = 512


@functools.cache
def _make_gather(B: int):
    rows_per_w = B // _NW
    chunks = rows_per_w // _CHUNK
    mesh = plsc.VectorSubcoreMesh(core_axis_name="c", subcore_axis_name="s")

    @functools.partial(
        pl.kernel,
        out_type=jax.ShapeDtypeStruct((B, _D), jnp.float32),
        mesh=mesh,
        scratch_types=[
            pltpu.VMEM((_CHUNK,), jnp.int32),
            pltpu.VMEM((_CHUNK, _D), jnp.float32),
            pltpu.SemaphoreType.DMA,
        ],
        compiler_params=pltpu.CompilerParams(use_tc_tiling_on_sc=False),
    )
    def gather_kernel(table_hbm, idx_hbm, out_hbm, idx_v, rows_v, sem):
        wid = lax.axis_index("s") * _NC + lax.axis_index("c")
        base = wid * rows_per_w

        @pl.loop(0, chunks)
        def _(c):
            off = base + c * _CHUNK
            pltpu.sync_copy(idx_hbm.at[pl.ds(off, _CHUNK)], idx_v)
            pltpu.async_copy(table_hbm.at[idx_v], rows_v, sem).wait()
            pltpu.sync_copy(rows_v, out_hbm.at[pl.ds(off, _CHUNK)])

    return gather_kernel


def kernel(x, weight):
    B = x.shape[0] * x.shape[1]
    flat = x.reshape(B)
    out = _make_gather(B)(weight, flat)
    return out.reshape(x.shape[0], x.shape[1], _D)


# speedup vs baseline: 1.8601x; 1.0348x over previous
import functools

import jax
import jax.numpy as jnp
from jax import lax
from jax.experimental import pallas as pl
from jax.experimental.pallas import tpu as pltpu
from jax.experimental.pallas import tpu_sc as plsc

_D = 64
_NC = 2
_NS = 16
_NW = _NC * _NS
_CHUNK = 512


@functools.cache
def _make_gather(B: int):
    rows_per_w = B // _NW
    chunks = rows_per_w // _CHUNK
    assert chunks % 2 == 0
    mesh = plsc.VectorSubcoreMesh(core_axis_name="c", subcore_axis_name="s")

    @functools.partial(
        pl.kernel,
        out_type=jax.ShapeDtypeStruct((B, _D), jnp.float32),
        mesh=mesh,
        scratch_types=[
            pltpu.VMEM((rows_per_w,), jnp.int32),
            pltpu.VMEM((2, _CHUNK, _D), jnp.float32),
            pltpu.SemaphoreType.DMA,
            pltpu.SemaphoreType.DMA,
        ],
        compiler_params=pltpu.CompilerParams(use_tc_tiling_on_sc=False),
    )
    def gather_kernel(table_hbm, idx_hbm, out_hbm, idx_v, rows_v, sem0, sem1):
        wid = lax.axis_index("s") * _NC + lax.axis_index("c")
        base = wid * rows_per_w
        pltpu.sync_copy(idx_hbm.at[pl.ds(base, rows_per_w)], idx_v)

        @pl.loop(0, chunks, step=2)
        def _(c):
            g0 = pltpu.async_copy(
                table_hbm.at[idx_v.at[pl.ds(c * _CHUNK, _CHUNK)]], rows_v.at[0], sem0)
            g1 = pltpu.async_copy(
                table_hbm.at[idx_v.at[pl.ds((c + 1) * _CHUNK, _CHUNK)]], rows_v.at[1], sem1)
            g0.wait()
            pltpu.sync_copy(rows_v.at[0], out_hbm.at[pl.ds(base + c * _CHUNK, _CHUNK)])
            g1.wait()
            pltpu.sync_copy(rows_v.at[1], out_hbm.at[pl.ds(base + (c + 1) * _CHUNK, _CHUNK)])

    return gather_kernel


def kernel(x, weight):
    B = x.shape[0] * x.shape[1]
    flat = x.reshape(B)
    out = _make_gather(B)(weight, flat)
    return out.reshape(x.shape[0], x.shape[1], _D)


# speedup vs baseline: 1.8678x; 1.0042x over previous
import functools

import jax
import jax.numpy as jnp
from jax import lax
from jax.experimental import pallas as pl
from jax.experimental.pallas import tpu as pltpu
from jax.experimental.pallas import tpu_sc as plsc

_D = 64
_NC = 2
_NS = 16
_NW = _NC * _NS
_CHUNK = 256
_NB = 4


@functools.cache
def _make_gather(B: int):
    rows_per_w = B // _NW
    chunks = rows_per_w // _CHUNK
    assert chunks % _NB == 0 and chunks > _NB
    mesh = plsc.VectorSubcoreMesh(core_axis_name="c", subcore_axis_name="s")

    @functools.partial(
        pl.kernel,
        out_type=jax.ShapeDtypeStruct((B, _D), jnp.float32),
        mesh=mesh,
        scratch_types=[
            pltpu.VMEM((rows_per_w,), jnp.int32),
            pltpu.VMEM((_NB, _CHUNK, _D), jnp.float32),
        ]
        + [pltpu.SemaphoreType.DMA] * (2 * _NB),
        compiler_params=pltpu.CompilerParams(use_tc_tiling_on_sc=False),
    )
    def gather_kernel(table_hbm, idx_hbm, out_hbm, idx_v, rows_v, *sems):
        gsem = sems[:_NB]
        wsem = sems[_NB:]
        wid = lax.axis_index("s") * _NC + lax.axis_index("c")
        base = wid * rows_per_w
        pltpu.sync_copy(idx_hbm.at[pl.ds(base, rows_per_w)], idx_v)

        def start_gather(c, b):
            pltpu.async_copy(
                table_hbm.at[idx_v.at[pl.ds(c * _CHUNK, _CHUNK)]],
                rows_v.at[b], gsem[b])

        def drain_gather(c, b):
            pltpu.make_async_copy(
                table_hbm.at[idx_v.at[pl.ds(c * _CHUNK, _CHUNK)]],
                rows_v.at[b], gsem[b]).wait()

        def start_write(c, b):
            return pltpu.async_copy(
                rows_v.at[b], out_hbm.at[pl.ds(base + c * _CHUNK, _CHUNK)],
                wsem[b])

        for b in range(_NB):
            start_gather(b, b)

        @pl.loop(0, chunks - _NB, step=_NB)
        def _(c):
            writes = []
            for b in range(_NB):
                drain_gather(c + b, b)
                writes.append(start_write(c + b, b))
            for b in range(_NB):
                writes[b].wait()
                start_gather(c + _NB + b, b)

        for b in range(_NB):
            c = chunks - _NB + b
            drain_gather(c, b)
            start_write(c, b).wait()

    return gather_kernel


def kernel(x, weight):
    B = x.shape[0] * x.shape[1]
    flat = x.reshape(B)
    out = _make_gather(B)(weight, flat)
    return out.reshape(x.shape[0], x.shape[1], _D)


# speedup vs baseline: 1.8713x; 1.0019x over previous
import functools

import jax
import jax.numpy as jnp
from jax import lax
from jax.experimental import pallas as pl
from jax.experimental.pallas import tpu as pltpu
from jax.experimental.pallas import tpu_sc as plsc

_D = 64
_NC = 2
_NS = 16
_NW = _NC * _NS
_CHUNK = 256
_NB = 5


@functools.cache
def _make_gather(B: int):
    rows_per_w = B // _NW
    chunks = rows_per_w // _CHUNK
    assert chunks % _NB == 0 and chunks > _NB
    mesh = plsc.VectorSubcoreMesh(core_axis_name="c", subcore_axis_name="s")

    @functools.partial(
        pl.kernel,
        out_type=jax.ShapeDtypeStruct((B, _D), jnp.float32),
        mesh=mesh,
        scratch_types=[
            pltpu.VMEM((rows_per_w,), jnp.int32),
            pltpu.VMEM((_NB, _CHUNK, _D), jnp.float32),
        ]
        + [pltpu.SemaphoreType.DMA] * (2 * _NB),
        compiler_params=pltpu.CompilerParams(use_tc_tiling_on_sc=False),
    )
    def gather_kernel(table_hbm, idx_hbm, out_hbm, idx_v, rows_v, *sems):
        gsem = sems[:_NB]
        wsem = sems[_NB:]
        wid = lax.axis_index("s") * _NC + lax.axis_index("c")
        base = wid * rows_per_w
        pltpu.sync_copy(idx_hbm.at[pl.ds(base, rows_per_w)], idx_v)

        def start_gather(c, b):
            pltpu.async_copy(
                table_hbm.at[idx_v.at[pl.ds(c * _CHUNK, _CHUNK)]],
                rows_v.at[b], gsem[b])

        def drain_gather(c, b):
            pltpu.make_async_copy(
                table_hbm.at[idx_v.at[pl.ds(c * _CHUNK, _CHUNK)]],
                rows_v.at[b], gsem[b]).wait()

        def start_write(c, b):
            return pltpu.async_copy(
                rows_v.at[b], out_hbm.at[pl.ds(base + c * _CHUNK, _CHUNK)],
                wsem[b])

        for b in range(_NB):
            start_gather(b, b)

        @pl.loop(0, chunks - _NB, step=_NB)
        def _(c):
            writes = []
            for b in range(_NB):
                drain_gather(c + b, b)
                writes.append(start_write(c + b, b))
            for b in range(_NB):
                writes[b].wait()
                start_gather(c + _NB + b, b)

        for b in range(_NB):
            c = chunks - _NB + b
            drain_gather(c, b)
            start_write(c, b).wait()

    return gather_kernel


def kernel(x, weight):
    B = x.shape[0] * x.shape[1]
    flat = x.reshape(B)
    out = _make_gather(B)(weight, flat)
    return out.reshape(x.shape[0], x.shape[1], _D)
